# baseline (speedup 1.0000x reference)
import jax
import jax.numpy as jnp
from jax import lax
from jax.experimental import pallas as pl
from jax.experimental.pallas import tpu as pltpu
from jax.experimental.pallas import tpu_sc as plsc

N = 10000
E = 160000
F = 256
D = 256
H = 512
R = 4
B = 32

HALF = 128
NSUB = 16
EPW = E // NSUB
CH = 80
NCH = EPW // CH
NMC = 25
MCH = EPW // NMC
MNCH = MCH // CH
NTRIP = NMC // 3
NPAD = 10240
NPN = NPAD // NSUB

TN1 = 2000
TN3 = 2000


def _k1_body(x_ref, wrel_ref, wroot_ref, b_ref, xw_ref, root_ref):
    xt = x_ref[...]
    for r in range(R):
        xw_ref[r] = jnp.dot(xt, wrel_ref[r], preferred_element_type=jnp.float32)
    root_ref[...] = (
        jnp.dot(xt, wroot_ref[...], preferred_element_type=jnp.float32) + b_ref[...]
    )


_k1 = pl.pallas_call(
    _k1_body,
    grid=(N // TN1,),
    in_specs=[
        pl.BlockSpec((TN1, F), lambda i: (i, 0)),
        pl.BlockSpec((R, F, D), lambda i: (0, 0, 0)),
        pl.BlockSpec((F, D), lambda i: (0, 0)),
        pl.BlockSpec((1, D), lambda i: (0, 0)),
    ],
    out_specs=[
        pl.BlockSpec((R, TN1, D), lambda i: (0, i, 0)),
        pl.BlockSpec((TN1, D), lambda i: (i, 0)),
    ],
    out_shape=[
        jax.ShapeDtypeStruct((R, N, D), jnp.float32),
        jax.ShapeDtypeStruct((N, D), jnp.float32),
    ],
)


def _sc_body(table, eif, et1, agg_out, degp_out,
             es0, ee0, ed0, es1, ee1, ed1, es2, ee2, ed2,
             gidx0, gidx1, gidx2, didx0, didx1, didx2,
             buf0, buf1, buf2, deg_v, agg_sh,
             sg0, sg1, sg2, ss0, ss1, ss2, si0, si1, si2):
    c = lax.axis_index("c")
    s = lax.axis_index("s")
    zero16 = jnp.zeros((16,), jnp.float32)
    one16 = jnp.ones((16,), jnp.float32)

    eidx = ((es0, ee0, ed0), (es1, ee1, ed1), (es2, ee2, ed2))
    gidx = (gidx0, gidx1, gidx2)
    didx = (didx0, didx1, didx2)
    buf = (buf0, buf1, buf2)
    semg = (sg0, sg1, sg2)
    sems = (ss0, ss1, ss2)
    semi = (si0, si1, si2)
    ebase = s * EPW

    def _stage(m, k):
        off = ebase + m * MCH
        pltpu.async_copy(eif.at[pl.ds(off, MCH)], eidx[k][0], semi[k])
        pltpu.async_copy(et1.at[pl.ds(off, MCH)], eidx[k][1], semi[k])
        pltpu.async_copy(eif.at[pl.ds(E + off, MCH)], eidx[k][2], semi[k])

    def _stage_wait(m, k):
        off = ebase + m * MCH
        pltpu.make_async_copy(eif.at[pl.ds(off, MCH)], eidx[k][0], semi[k]).wait()
        pltpu.make_async_copy(et1.at[pl.ds(off, MCH)], eidx[k][1], semi[k]).wait()
        pltpu.make_async_copy(eif.at[pl.ds(E + off, MCH)], eidx[k][2], semi[k]).wait()

    for k in range(3):
        _stage(k, k)

    def _z1(i, carry):
        deg_v[pl.ds(i * 16, 16)] = zero16
        return carry

    lax.fori_loop(0, N // 16, _z1, 0)

    def _z2(i, carry):
        for j in range(HALF // 16):
            buf0[i, pl.ds(j * 16, 16)] = zero16
        return carry

    lax.fori_loop(0, CH, _z2, 0)

    nbase = s * NPN
    for j in range(NPN // CH):
        pltpu.sync_copy(buf0, agg_sh.at[pl.ds(nbase + j * CH, CH)])

    plsc.subcore_barrier()

    def _prep(km, jloc, k, m):
        eb = eidx[km]
        for i in range(CH // 16):
            sl = pl.ds(jloc * CH + i * 16, 16)
            dv = eb[2][sl]
            gidx[k][pl.ds(i * 16, 16)] = (eb[1][sl] * N + eb[0][sl]) * 2 + c
            didx[k][pl.ds(i * 16, 16)] = dv

            @pl.when(c == 0)
            def _deg():
                plsc.addupdate_scatter(deg_v, [dv], one16)

    def _wait_scat(k):
        pltpu.make_async_copy(buf[k], agg_sh.at[didx[k]], sems[k]).wait()

    def _chunk(km, jloc, k, wait_scat, prev, m):
        if wait_scat:
            _wait_scat(k)
        _prep(km, jloc, k, m)
        pltpu.async_copy(table.at[gidx[k]], buf[k], semg[k])
        if prev is not None:
            kp = prev
            pltpu.make_async_copy(table.at[gidx[kp]], buf[kp], semg[kp]).wait()
            pltpu.async_copy(buf[kp], agg_sh.at[didx[kp]], sems[kp], add=True)

    def _triple(t, first):
        for mi in range(3):
            m = 3 * t + mi
            _stage_wait(m, mi)
            for jloc in range(MNCH):
                ci = 5 * mi + jloc
                k = ci % 3
                skip_ws = first and ci < 3
                skip_prev = first and ci == 0
                _chunk(mi, jloc, k,
                       wait_scat=not skip_ws,
                       prev=None if skip_prev else (k + 2) % 3,
                       m=m)
            if first:
                _stage(m + 3, mi)
            else:
                @pl.when(m + 3 < NMC)
                def _pf():
                    _stage(m + 3, mi)

    _triple(0, True)

    def _tloop(t, carry):
        _triple(t, False)
        return carry

    lax.fori_loop(1, NTRIP, _tloop, 0)

    _stage_wait(NMC - 1, 0)
    for jloc in range(MNCH):
        k = jloc % 3
        _chunk(0, jloc, k, wait_scat=True, prev=(k + 2) % 3, m=NMC - 1)

    pltpu.make_async_copy(table.at[gidx[1]], buf[1], semg[1]).wait()
    pltpu.async_copy(buf[1], agg_sh.at[didx[1]], sems[1], add=True)
    for k in range(3):
        _wait_scat(k)

    plsc.subcore_barrier()

    pltpu.sync_copy(agg_sh.at[pl.ds(nbase, NPN)], agg_out.at[c, pl.ds(nbase, NPN)])

    @pl.when(c == 0)
    def _degout():
        pltpu.sync_copy(deg_v, degp_out.at[pl.ds(s * N, N)])


import functools


@functools.lru_cache(maxsize=1)
def _get_k2():
  _sc_mesh = plsc.VectorSubcoreMesh(
      core_axis_name="c", subcore_axis_name="s", num_cores=2, num_subcores=NSUB
  )
  return pl.kernel(
    _sc_body,
    out_type=[
        jax.ShapeDtypeStruct((2, NPAD, HALF), jnp.float32),
        jax.ShapeDtypeStruct((NSUB * N,), jnp.float32),
    ],
    mesh=_sc_mesh,
    compiler_params=pltpu.CompilerParams(needs_layout_passes=False),
    scratch_types=(
        [pltpu.VMEM((MCH,), jnp.int32)] * 9
        + [pltpu.VMEM((CH,), jnp.int32)] * 6
        + [pltpu.VMEM((CH, HALF), jnp.float32)] * 3
        + [pltpu.VMEM((N,), jnp.float32)]
        + [pltpu.VMEM_SHARED((NPAD, HALF), jnp.float32)]
        + [pltpu.SemaphoreType.DMA] * 9
    ),
  )


def _k3_body(msg_ref, wfc_ref, bfc_ref, agg_ref, degt_ref, root_ref, gum_ref,
             out_ref, samp_ref, ent_ref,
             msgr, m_run, z_run, s1_run, m2_run, am_run):
    i = pl.program_id(0)

    @pl.when(i == 0)
    def _():
        msgr[...] = (
            lax.dot_general(
                msg_ref[...], wfc_ref[...], (((1,), (1,)), ((), ())),
                preferred_element_type=jnp.float32,
            )
            + bfc_ref[...]
        )
        m_run[...] = jnp.full((1, B), -3e38, jnp.float32)
        z_run[...] = jnp.zeros((1, B), jnp.float32)
        s1_run[...] = jnp.zeros((1, B), jnp.float32)
        m2_run[...] = jnp.full((1, B), -3e38, jnp.float32)
        am_run[...] = jnp.zeros((1, B), jnp.int32)

    deg = jnp.sum(degt_ref[...], axis=1, keepdims=True)
    inv = 1.0 / jnp.maximum(deg, 1.0)
    m = msgr[...]
    acc = None
    for h in range(2):
        emb = jnp.maximum(
            agg_ref[h] * inv + root_ref[:, h * HALF:(h + 1) * HALF], 0.0
        )
        part = lax.dot_general(
            emb, m[:, h * HALF:(h + 1) * HALF], (((1,), (1,)), ((), ())),
            preferred_element_type=jnp.float32,
        )
        acc = part if acc is None else acc + part
    out_ref[...] = acc

    tm = jnp.max(acc, axis=0, keepdims=True)
    m_new = jnp.maximum(m_run[...], tm)
    dm = m_run[...] - m_new
    scale = jnp.exp(dm)
    e_t = jnp.exp(acc - m_new)
    s1_run[...] = (
        scale * (s1_run[...] + dm * z_run[...])
        + jnp.sum(e_t * (acc - m_new), axis=0, keepdims=True)
    )
    z_run[...] = z_run[...] * scale + jnp.sum(e_t, axis=0, keepdims=True)
    m_run[...] = m_new

    tz = acc + gum_ref[...]
    tm2 = jnp.max(tz, axis=0, keepdims=True)
    iota = lax.broadcasted_iota(jnp.int32, (TN3, B), 0) + i * TN3
    idx_t = jnp.min(jnp.where(tz == tm2, iota, jnp.int32(N)), axis=0, keepdims=True)
    better = tm2 > m2_run[...]
    m2_run[...] = jnp.where(better, tm2, m2_run[...])
    am_run[...] = jnp.where(better, idx_t, am_run[...])

    @pl.when(i == (N // TN3) - 1)
    def _fin():
        z = z_run[...]
        ent_ref[...] = jnp.log(z) - s1_run[...] / z
        samp_ref[...] = am_run[...]


_k3 = pl.pallas_call(
    _k3_body,
    grid=(N // TN3,),
    in_specs=[
        pl.BlockSpec((B, H), lambda i: (0, 0)),
        pl.BlockSpec((D, H), lambda i: (0, 0)),
        pl.BlockSpec((1, D), lambda i: (0, 0)),
        pl.BlockSpec((2, TN3, HALF), lambda i: (0, i, 0)),
        pl.BlockSpec((TN3, NSUB), lambda i: (i, 0)),
        pl.BlockSpec((TN3, D), lambda i: (i, 0)),
        pl.BlockSpec((TN3, B), lambda i: (i, 0)),
    ],
    out_specs=[
        pl.BlockSpec((TN3, B), lambda i: (i, 0)),
        pl.BlockSpec((1, B), lambda i: (0, 0)),
        pl.BlockSpec((1, B), lambda i: (0, 0)),
    ],
    out_shape=[
        jax.ShapeDtypeStruct((N, B), jnp.float32),
        jax.ShapeDtypeStruct((1, B), jnp.int32),
        jax.ShapeDtypeStruct((1, B), jnp.float32),
    ],
    scratch_shapes=[
        pltpu.VMEM((B, D), jnp.float32),
        pltpu.VMEM((1, B), jnp.float32),
        pltpu.VMEM((1, B), jnp.float32),
        pltpu.VMEM((1, B), jnp.float32),
        pltpu.VMEM((1, B), jnp.float32),
        pltpu.VMEM((1, B), jnp.int32),
    ],
)


@functools.lru_cache(maxsize=1)
def _gumbel_t():
    u = jax.random.uniform(jax.random.key(42), (B, N), minval=1e-20, maxval=1.0)
    return (-jnp.log(-jnp.log(u))).T


def kernel(message, x, edge_index, edge_type, W_rel, W_root, b, W_fc, b_fc):
    eif = edge_index.astype(jnp.int32).reshape(2 * E)
    et = edge_type.astype(jnp.int32)

    xw, root = _k1(x, W_rel, W_root, b.reshape(1, D))
    table = xw.reshape(R * N * 2, HALF)

    agg2, degp = _get_k2()(table, eif, et)
    degt = degp.reshape(NSUB, N).T

    logits_t, samp, ent = _k3(
        message, W_fc, b_fc.reshape(1, D), agg2, degt, root, _gumbel_t()
    )
    return samp.reshape(B), logits_t.T, ent.reshape(B)


# speedup vs baseline: 1.0096x; 1.0096x over previous
import jax
import jax.numpy as jnp
from jax import lax
from jax.experimental import pallas as pl
from jax.experimental.pallas import tpu as pltpu
from jax.experimental.pallas import tpu_sc as plsc

N = 10000
E = 160000
F = 256
D = 256
H = 512
R = 4
B = 32

HALF = 128
NSUB = 16
EPW = E // NSUB
CH = 80
NCH = EPW // CH
NMC = 25
MCH = EPW // NMC
MNCH = MCH // CH
NTRIP = NMC // 3
NPAD = 10240
NPN = NPAD // NSUB

TN1 = 2000
TN3 = 2000


def _k1_body(x_ref, wrel_ref, xw_ref):
    xt = x_ref[...]
    for r in range(R):
        xw_ref[r] = jnp.dot(xt, wrel_ref[r], preferred_element_type=jnp.float32)


_k1 = pl.pallas_call(
    _k1_body,
    grid=(N // TN1,),
    in_specs=[
        pl.BlockSpec((TN1, F), lambda i: (i, 0)),
        pl.BlockSpec((R, F, D), lambda i: (0, 0, 0)),
    ],
    out_specs=pl.BlockSpec((R, TN1, D), lambda i: (0, i, 0)),
    out_shape=jax.ShapeDtypeStruct((R, N, D), jnp.float32),
)


def _sc_body(table, eif, et1, agg_out, degp_out,
             es0, ee0, ed0, es1, ee1, ed1, es2, ee2, ed2,
             gidx0, gidx1, gidx2, didx0, didx1, didx2,
             buf0, buf1, buf2, deg_v, agg_sh,
             sg0, sg1, sg2, ss0, ss1, ss2, si0, si1, si2):
    c = lax.axis_index("c")
    s = lax.axis_index("s")
    zero16 = jnp.zeros((16,), jnp.float32)
    one16 = jnp.ones((16,), jnp.float32)

    eidx = ((es0, ee0, ed0), (es1, ee1, ed1), (es2, ee2, ed2))
    gidx = (gidx0, gidx1, gidx2)
    didx = (didx0, didx1, didx2)
    buf = (buf0, buf1, buf2)
    semg = (sg0, sg1, sg2)
    sems = (ss0, ss1, ss2)
    semi = (si0, si1, si2)
    ebase = s * EPW

    def _stage(m, k):
        off = ebase + m * MCH
        pltpu.async_copy(eif.at[pl.ds(off, MCH)], eidx[k][0], semi[k])
        pltpu.async_copy(et1.at[pl.ds(off, MCH)], eidx[k][1], semi[k])
        pltpu.async_copy(eif.at[pl.ds(E + off, MCH)], eidx[k][2], semi[k])

    def _stage_wait(m, k):
        off = ebase + m * MCH
        pltpu.make_async_copy(eif.at[pl.ds(off, MCH)], eidx[k][0], semi[k]).wait()
        pltpu.make_async_copy(et1.at[pl.ds(off, MCH)], eidx[k][1], semi[k]).wait()
        pltpu.make_async_copy(eif.at[pl.ds(E + off, MCH)], eidx[k][2], semi[k]).wait()

    for k in range(3):
        _stage(k, k)

    def _z1(i, carry):
        deg_v[pl.ds(i * 16, 16)] = zero16
        return carry

    lax.fori_loop(0, N // 16, _z1, 0)

    def _z2(i, carry):
        for j in range(HALF // 16):
            buf0[i, pl.ds(j * 16, 16)] = zero16
        return carry

    lax.fori_loop(0, CH, _z2, 0)

    nbase = s * NPN
    for j in range(NPN // CH):
        pltpu.sync_copy(buf0, agg_sh.at[pl.ds(nbase + j * CH, CH)])

    plsc.subcore_barrier()

    def _prep(km, jloc, k, m):
        eb = eidx[km]
        for i in range(CH // 16):
            sl = pl.ds(jloc * CH + i * 16, 16)
            dv = eb[2][sl]
            gidx[k][pl.ds(i * 16, 16)] = (eb[1][sl] * N + eb[0][sl]) * 2 + c
            didx[k][pl.ds(i * 16, 16)] = dv

            @pl.when(c == 0)
            def _deg():
                plsc.addupdate_scatter(deg_v, [dv], one16)

    def _wait_scat(k):
        pltpu.make_async_copy(buf[k], agg_sh.at[didx[k]], sems[k]).wait()

    def _chunk(km, jloc, k, wait_scat, prev, m):
        if wait_scat:
            _wait_scat(k)
        _prep(km, jloc, k, m)
        pltpu.async_copy(table.at[gidx[k]], buf[k], semg[k])
        if prev is not None:
            kp = prev
            pltpu.make_async_copy(table.at[gidx[kp]], buf[kp], semg[kp]).wait()
            pltpu.async_copy(buf[kp], agg_sh.at[didx[kp]], sems[kp], add=True)

    def _triple(t, first):
        for mi in range(3):
            m = 3 * t + mi
            _stage_wait(m, mi)
            for jloc in range(MNCH):
                ci = 5 * mi + jloc
                k = ci % 3
                skip_ws = first and ci < 3
                skip_prev = first and ci == 0
                _chunk(mi, jloc, k,
                       wait_scat=not skip_ws,
                       prev=None if skip_prev else (k + 2) % 3,
                       m=m)
            if first:
                _stage(m + 3, mi)
            else:
                @pl.when(m + 3 < NMC)
                def _pf():
                    _stage(m + 3, mi)

    _triple(0, True)

    def _tloop(t, carry):
        _triple(t, False)
        return carry

    lax.fori_loop(1, NTRIP, _tloop, 0)

    _stage_wait(NMC - 1, 0)
    for jloc in range(MNCH):
        k = jloc % 3
        _chunk(0, jloc, k, wait_scat=True, prev=(k + 2) % 3, m=NMC - 1)

    pltpu.make_async_copy(table.at[gidx[1]], buf[1], semg[1]).wait()
    pltpu.async_copy(buf[1], agg_sh.at[didx[1]], sems[1], add=True)
    for k in range(3):
        _wait_scat(k)

    plsc.subcore_barrier()

    pltpu.sync_copy(agg_sh.at[pl.ds(nbase, NPN)], agg_out.at[c, pl.ds(nbase, NPN)])

    @pl.when(c == 0)
    def _degout():
        pltpu.sync_copy(deg_v, degp_out.at[pl.ds(s * N, N)])


import functools


@functools.lru_cache(maxsize=1)
def _get_k2():
  _sc_mesh = plsc.VectorSubcoreMesh(
      core_axis_name="c", subcore_axis_name="s", num_cores=2, num_subcores=NSUB
  )
  return pl.kernel(
    _sc_body,
    out_type=[
        jax.ShapeDtypeStruct((2, NPAD, HALF), jnp.float32),
        jax.ShapeDtypeStruct((NSUB * N,), jnp.float32),
    ],
    mesh=_sc_mesh,
    compiler_params=pltpu.CompilerParams(needs_layout_passes=False),
    scratch_types=(
        [pltpu.VMEM((MCH,), jnp.int32)] * 9
        + [pltpu.VMEM((CH,), jnp.int32)] * 6
        + [pltpu.VMEM((CH, HALF), jnp.float32)] * 3
        + [pltpu.VMEM((N,), jnp.float32)]
        + [pltpu.VMEM_SHARED((NPAD, HALF), jnp.float32)]
        + [pltpu.SemaphoreType.DMA] * 9
    ),
  )


def _k3_body(msg_ref, wfc_ref, bfc_ref, agg_ref, degt_ref, x_ref, wroot_ref,
             b_ref, gum_ref, out_ref, samp_ref, ent_ref,
             msgr, m_run, z_run, s1_run, m2_run, am_run):
    i = pl.program_id(0)

    @pl.when(i == 0)
    def _():
        msgr[...] = (
            lax.dot_general(
                msg_ref[...], wfc_ref[...], (((1,), (1,)), ((), ())),
                preferred_element_type=jnp.float32,
            )
            + bfc_ref[...]
        )
        m_run[...] = jnp.full((1, B), -3e38, jnp.float32)
        z_run[...] = jnp.zeros((1, B), jnp.float32)
        s1_run[...] = jnp.zeros((1, B), jnp.float32)
        m2_run[...] = jnp.full((1, B), -3e38, jnp.float32)
        am_run[...] = jnp.zeros((1, B), jnp.int32)

    deg = jnp.sum(degt_ref[...], axis=1, keepdims=True)
    inv = 1.0 / jnp.maximum(deg, 1.0)
    m = msgr[...]
    root = (
        jnp.dot(x_ref[...], wroot_ref[...], preferred_element_type=jnp.float32)
        + b_ref[...]
    )
    acc = None
    for h in range(2):
        emb = jnp.maximum(
            agg_ref[h] * inv + root[:, h * HALF:(h + 1) * HALF], 0.0
        )
        part = lax.dot_general(
            emb, m[:, h * HALF:(h + 1) * HALF], (((1,), (1,)), ((), ())),
            preferred_element_type=jnp.float32,
        )
        acc = part if acc is None else acc + part
    out_ref[...] = acc

    tm = jnp.max(acc, axis=0, keepdims=True)
    m_new = jnp.maximum(m_run[...], tm)
    dm = m_run[...] - m_new
    scale = jnp.exp(dm)
    e_t = jnp.exp(acc - m_new)
    s1_run[...] = (
        scale * (s1_run[...] + dm * z_run[...])
        + jnp.sum(e_t * (acc - m_new), axis=0, keepdims=True)
    )
    z_run[...] = z_run[...] * scale + jnp.sum(e_t, axis=0, keepdims=True)
    m_run[...] = m_new

    tz = acc + gum_ref[...]
    tm2 = jnp.max(tz, axis=0, keepdims=True)
    iota = lax.broadcasted_iota(jnp.int32, (TN3, B), 0) + i * TN3
    idx_t = jnp.min(jnp.where(tz == tm2, iota, jnp.int32(N)), axis=0, keepdims=True)
    better = tm2 > m2_run[...]
    m2_run[...] = jnp.where(better, tm2, m2_run[...])
    am_run[...] = jnp.where(better, idx_t, am_run[...])

    @pl.when(i == (N // TN3) - 1)
    def _fin():
        z = z_run[...]
        ent_ref[...] = jnp.log(z) - s1_run[...] / z
        samp_ref[...] = am_run[...]


_k3 = pl.pallas_call(
    _k3_body,
    grid=(N // TN3,),
    in_specs=[
        pl.BlockSpec((B, H), lambda i: (0, 0)),
        pl.BlockSpec((D, H), lambda i: (0, 0)),
        pl.BlockSpec((1, D), lambda i: (0, 0)),
        pl.BlockSpec((2, TN3, HALF), lambda i: (0, i, 0)),
        pl.BlockSpec((TN3, NSUB), lambda i: (i, 0)),
        pl.BlockSpec((TN3, F), lambda i: (i, 0)),
        pl.BlockSpec((F, D), lambda i: (0, 0)),
        pl.BlockSpec((1, D), lambda i: (0, 0)),
        pl.BlockSpec((TN3, B), lambda i: (i, 0)),
    ],
    out_specs=[
        pl.BlockSpec((TN3, B), lambda i: (i, 0)),
        pl.BlockSpec((1, B), lambda i: (0, 0)),
        pl.BlockSpec((1, B), lambda i: (0, 0)),
    ],
    out_shape=[
        jax.ShapeDtypeStruct((N, B), jnp.float32),
        jax.ShapeDtypeStruct((1, B), jnp.int32),
        jax.ShapeDtypeStruct((1, B), jnp.float32),
    ],
    scratch_shapes=[
        pltpu.VMEM((B, D), jnp.float32),
        pltpu.VMEM((1, B), jnp.float32),
        pltpu.VMEM((1, B), jnp.float32),
        pltpu.VMEM((1, B), jnp.float32),
        pltpu.VMEM((1, B), jnp.float32),
        pltpu.VMEM((1, B), jnp.int32),
    ],
)


@functools.lru_cache(maxsize=1)
def _gumbel_t():
    u = jax.random.uniform(jax.random.key(42), (B, N), minval=1e-20, maxval=1.0)
    return (-jnp.log(-jnp.log(u))).T


def kernel(message, x, edge_index, edge_type, W_rel, W_root, b, W_fc, b_fc):
    eif = edge_index.astype(jnp.int32).reshape(2 * E)
    et = edge_type.astype(jnp.int32)

    xw = _k1(x, W_rel)
    table = xw.reshape(R * N * 2, HALF)

    agg2, degp = _get_k2()(table, eif, et)
    degt = degp.reshape(NSUB, N).T

    logits_t, samp, ent = _k3(
        message, W_fc, b_fc.reshape(1, D), agg2, degt, x, W_root,
        b.reshape(1, D), _gumbel_t()
    )
    return samp.reshape(B), logits_t.T, ent.reshape(B)


# speedup vs baseline: 1.0324x; 1.0226x over previous
import jax
import jax.numpy as jnp
from jax import lax
from jax.experimental import pallas as pl
from jax.experimental.pallas import tpu as pltpu
from jax.experimental.pallas import tpu_sc as plsc

N = 10000
E = 160000
F = 256
D = 256
H = 512
R = 4
B = 32

HALF = 128
NSUB = 16
EPW = E // NSUB
CH = 80
NCH = EPW // CH
NMC = 25
MCH = EPW // NMC
MNCH = MCH // CH
NTRIP = NMC // 3
NPAD = 10240
NPN = NPAD // NSUB

TN1 = 2000
TN3 = 2000


def _k1_body(x_ref, wrel_ref, xw_ref):
    xt = x_ref[...]
    for r in range(R):
        xw_ref[r] = jnp.dot(xt, wrel_ref[r], preferred_element_type=jnp.float32)


_k1 = pl.pallas_call(
    _k1_body,
    grid=(N // TN1,),
    in_specs=[
        pl.BlockSpec((TN1, F), lambda i: (i, 0)),
        pl.BlockSpec((R, F, D), lambda i: (0, 0, 0)),
    ],
    out_specs=pl.BlockSpec((R, TN1, D), lambda i: (0, i, 0)),
    out_shape=jax.ShapeDtypeStruct((R, N, D), jnp.float32),
)


def _sc_body(table, eif, et1, agg_out, degp_out,
             es0, ee0, ed0, es1, ee1, ed1, es2, ee2, ed2,
             gidx0, gidx1, gidx2, didx0, didx1, didx2,
             buf0, buf1, buf2, deg_v, agg_sh,
             sg0, sg1, sg2, ss0, ss1, ss2, si0, si1, si2):
    c = lax.axis_index("c")
    s = lax.axis_index("s")
    zero16 = jnp.zeros((16,), jnp.float32)
    one16 = jnp.ones((16,), jnp.float32)

    eidx = ((es0, ee0, ed0), (es1, ee1, ed1), (es2, ee2, ed2))
    gidx = (gidx0, gidx1, gidx2)
    didx = (didx0, didx1, didx2)
    buf = (buf0, buf1, buf2)
    semg = (sg0, sg1, sg2)
    sems = (ss0, ss1, ss2)
    semi = (si0, si1, si2)
    ebase = s * EPW

    def _stage(m, k):
        off = ebase + m * MCH
        pltpu.async_copy(eif.at[pl.ds(off, MCH)], eidx[k][0], semi[k])
        pltpu.async_copy(et1.at[pl.ds(off, MCH)], eidx[k][1], semi[k])
        pltpu.async_copy(eif.at[pl.ds(E + off, MCH)], eidx[k][2], semi[k])

    def _stage_wait(m, k):
        off = ebase + m * MCH
        pltpu.make_async_copy(eif.at[pl.ds(off, MCH)], eidx[k][0], semi[k]).wait()
        pltpu.make_async_copy(et1.at[pl.ds(off, MCH)], eidx[k][1], semi[k]).wait()
        pltpu.make_async_copy(eif.at[pl.ds(E + off, MCH)], eidx[k][2], semi[k]).wait()

    for k in range(3):
        _stage(k, k)

    def _z1(i, carry):
        deg_v[pl.ds(i * 16, 16)] = zero16
        return carry

    lax.fori_loop(0, N // 16, _z1, 0)

    def _z2(i, carry):
        for j in range(HALF // 16):
            buf0[i, pl.ds(j * 16, 16)] = zero16
        return carry

    lax.fori_loop(0, CH, _z2, 0)

    nbase = s * NPN
    for j in range(NPN // CH):
        pltpu.sync_copy(buf0, agg_sh.at[pl.ds(nbase + j * CH, CH)])

    plsc.subcore_barrier()

    def _prep(km, jloc, k, m):
        eb = eidx[km]
        for i in range(CH // 16):
            sl = pl.ds(jloc * CH + i * 16, 16)
            dv = eb[2][sl]
            gidx[k][pl.ds(i * 16, 16)] = (eb[1][sl] * N + eb[0][sl]) * 2 + c
            didx[k][pl.ds(i * 16, 16)] = dv

            @pl.when(c == 0)
            def _deg():
                plsc.addupdate_scatter(deg_v, [dv], one16)

    def _wait_scat(k):
        pltpu.make_async_copy(buf[k], agg_sh.at[didx[k]], sems[k]).wait()

    def _chunk(km, jloc, k, wait_scat, prev, m):
        if wait_scat:
            _wait_scat(k)
        _prep(km, jloc, k, m)
        pltpu.async_copy(table.at[gidx[k]], buf[k], semg[k])
        if prev is not None:
            kp = prev
            pltpu.make_async_copy(table.at[gidx[kp]], buf[kp], semg[kp]).wait()
            pltpu.async_copy(buf[kp], agg_sh.at[didx[kp]], sems[kp], add=True)

    def _triple(t, first):
        for mi in range(3):
            m = 3 * t + mi
            _stage_wait(m, mi)
            for jloc in range(MNCH):
                ci = 5 * mi + jloc
                k = ci % 3
                skip_ws = first and ci < 3
                skip_prev = first and ci < 2
                _chunk(mi, jloc, k,
                       wait_scat=not skip_ws,
                       prev=None if skip_prev else (k + 1) % 3,
                       m=m)
            if first:
                _stage(m + 3, mi)
            else:
                @pl.when(m + 3 < NMC)
                def _pf():
                    _stage(m + 3, mi)

    _triple(0, True)

    def _tloop(t, carry):
        _triple(t, False)
        return carry

    lax.fori_loop(1, NTRIP, _tloop, 0)

    _stage_wait(NMC - 1, 0)
    for jloc in range(MNCH):
        k = jloc % 3
        _chunk(0, jloc, k, wait_scat=True, prev=(k + 1) % 3, m=NMC - 1)

    for k in (0, 1):
        pltpu.make_async_copy(table.at[gidx[k]], buf[k], semg[k]).wait()
        pltpu.async_copy(buf[k], agg_sh.at[didx[k]], sems[k], add=True)
    for k in range(3):
        _wait_scat(k)

    plsc.subcore_barrier()

    pltpu.sync_copy(agg_sh.at[pl.ds(nbase, NPN)], agg_out.at[c, pl.ds(nbase, NPN)])

    @pl.when(c == 0)
    def _degout():
        pltpu.sync_copy(deg_v, degp_out.at[pl.ds(s * N, N)])


import functools


@functools.lru_cache(maxsize=1)
def _get_k2():
  _sc_mesh = plsc.VectorSubcoreMesh(
      core_axis_name="c", subcore_axis_name="s", num_cores=2, num_subcores=NSUB
  )
  return pl.kernel(
    _sc_body,
    out_type=[
        jax.ShapeDtypeStruct((2, NPAD, HALF), jnp.float32),
        jax.ShapeDtypeStruct((NSUB * N,), jnp.float32),
    ],
    mesh=_sc_mesh,
    compiler_params=pltpu.CompilerParams(needs_layout_passes=False),
    scratch_types=(
        [pltpu.VMEM((MCH,), jnp.int32)] * 9
        + [pltpu.VMEM((CH,), jnp.int32)] * 6
        + [pltpu.VMEM((CH, HALF), jnp.float32)] * 3
        + [pltpu.VMEM((N,), jnp.float32)]
        + [pltpu.VMEM_SHARED((NPAD, HALF), jnp.float32)]
        + [pltpu.SemaphoreType.DMA] * 9
    ),
  )


def _k3_body(msg_ref, wfc_ref, bfc_ref, agg_ref, degt_ref, x_ref, wroot_ref,
             b_ref, gum_ref, out_ref, samp_ref, ent_ref,
             msgr, m_run, z_run, s1_run, m2_run, am_run):
    i = pl.program_id(0)

    @pl.when(i == 0)
    def _():
        msgr[...] = (
            lax.dot_general(
                msg_ref[...], wfc_ref[...], (((1,), (1,)), ((), ())),
                preferred_element_type=jnp.float32,
            )
            + bfc_ref[...]
        )
        m_run[...] = jnp.full((1, B), -3e38, jnp.float32)
        z_run[...] = jnp.zeros((1, B), jnp.float32)
        s1_run[...] = jnp.zeros((1, B), jnp.float32)
        m2_run[...] = jnp.full((1, B), -3e38, jnp.float32)
        am_run[...] = jnp.zeros((1, B), jnp.int32)

    deg = jnp.sum(degt_ref[...], axis=1, keepdims=True)
    inv = 1.0 / jnp.maximum(deg, 1.0)
    m = msgr[...]
    root = (
        jnp.dot(x_ref[...], wroot_ref[...], preferred_element_type=jnp.float32)
        + b_ref[...]
    )
    acc = None
    for h in range(2):
        emb = jnp.maximum(
            agg_ref[h] * inv + root[:, h * HALF:(h + 1) * HALF], 0.0
        )
        part = lax.dot_general(
            emb, m[:, h * HALF:(h + 1) * HALF], (((1,), (1,)), ((), ())),
            preferred_element_type=jnp.float32,
        )
        acc = part if acc is None else acc + part
    out_ref[...] = acc

    tm = jnp.max(acc, axis=0, keepdims=True)
    m_new = jnp.maximum(m_run[...], tm)
    dm = m_run[...] - m_new
    scale = jnp.exp(dm)
    e_t = jnp.exp(acc - m_new)
    s1_run[...] = (
        scale * (s1_run[...] + dm * z_run[...])
        + jnp.sum(e_t * (acc - m_new), axis=0, keepdims=True)
    )
    z_run[...] = z_run[...] * scale + jnp.sum(e_t, axis=0, keepdims=True)
    m_run[...] = m_new

    tz = acc + gum_ref[...]
    tm2 = jnp.max(tz, axis=0, keepdims=True)
    iota = lax.broadcasted_iota(jnp.int32, (TN3, B), 0) + i * TN3
    idx_t = jnp.min(jnp.where(tz == tm2, iota, jnp.int32(N)), axis=0, keepdims=True)
    better = tm2 > m2_run[...]
    m2_run[...] = jnp.where(better, tm2, m2_run[...])
    am_run[...] = jnp.where(better, idx_t, am_run[...])

    @pl.when(i == (N // TN3) - 1)
    def _fin():
        z = z_run[...]
        ent_ref[...] = jnp.log(z) - s1_run[...] / z
        samp_ref[...] = am_run[...]


_k3 = pl.pallas_call(
    _k3_body,
    grid=(N // TN3,),
    in_specs=[
        pl.BlockSpec((B, H), lambda i: (0, 0)),
        pl.BlockSpec((D, H), lambda i: (0, 0)),
        pl.BlockSpec((1, D), lambda i: (0, 0)),
        pl.BlockSpec((2, TN3, HALF), lambda i: (0, i, 0)),
        pl.BlockSpec((TN3, NSUB), lambda i: (i, 0)),
        pl.BlockSpec((TN3, F), lambda i: (i, 0)),
        pl.BlockSpec((F, D), lambda i: (0, 0)),
        pl.BlockSpec((1, D), lambda i: (0, 0)),
        pl.BlockSpec((TN3, B), lambda i: (i, 0)),
    ],
    out_specs=[
        pl.BlockSpec((TN3, B), lambda i: (i, 0)),
        pl.BlockSpec((1, B), lambda i: (0, 0)),
        pl.BlockSpec((1, B), lambda i: (0, 0)),
    ],
    out_shape=[
        jax.ShapeDtypeStruct((N, B), jnp.float32),
        jax.ShapeDtypeStruct((1, B), jnp.int32),
        jax.ShapeDtypeStruct((1, B), jnp.float32),
    ],
    scratch_shapes=[
        pltpu.VMEM((B, D), jnp.float32),
        pltpu.VMEM((1, B), jnp.float32),
        pltpu.VMEM((1, B), jnp.float32),
        pltpu.VMEM((1, B), jnp.float32),
        pltpu.VMEM((1, B), jnp.float32),
        pltpu.VMEM((1, B), jnp.int32),
    ],
)


@functools.lru_cache(maxsize=1)
def _gumbel_t():
    u = jax.random.uniform(jax.random.key(42), (B, N), minval=1e-20, maxval=1.0)
    return (-jnp.log(-jnp.log(u))).T


def kernel(message, x, edge_index, edge_type, W_rel, W_root, b, W_fc, b_fc):
    eif = edge_index.astype(jnp.int32).reshape(2 * E)
    et = edge_type.astype(jnp.int32)

    xw = _k1(x, W_rel)
    table = xw.reshape(R * N * 2, HALF)

    agg2, degp = _get_k2()(table, eif, et)
    degt = degp.reshape(NSUB, N).T

    logits_t, samp, ent = _k3(
        message, W_fc, b_fc.reshape(1, D), agg2, degt, x, W_root,
        b.reshape(1, D), _gumbel_t()
    )
    return samp.reshape(B), logits_t.T, ent.reshape(B)
